# baseline (speedup 1.0000x reference)
import functools
import math

import jax
import jax.numpy as jnp
from jax import lax
from jax.experimental import pallas as pl
from jax.experimental.pallas import tpu as pltpu
from jax.experimental.pallas import tpu_sc as plsc

EPS = 1e-5
RS = 1.0 / math.sqrt(1.0 + EPS)
_SQRT_HALF = 1.0 / math.sqrt(2.0)

_NW = 32
_GROW = 128


def _gelu(v):
    return 0.5 * v * (1.0 + lax.erf(v * _SQRT_HALF))


def _dot(a, b):
    return jnp.dot(a, b, preferred_element_type=jnp.float32)


def _sc_gather(table, idx, chunk=400):
    total = idx.shape[0]
    width = table.shape[1]
    dtype = table.dtype
    epw = total // _NW
    assert epw % chunk == 0 and chunk % 8 == 0 and epw % 8 == 0
    steps = epw // chunk

    mesh = plsc.VectorSubcoreMesh(core_axis_name="c", subcore_axis_name="s")

    @functools.partial(
        pl.kernel,
        out_type=jax.ShapeDtypeStruct((total, width), dtype),
        mesh=mesh,
        scratch_types=[
            pltpu.VMEM((epw,), jnp.int32),
            pltpu.VMEM((2, chunk, width), dtype),
            pltpu.SemaphoreType.DMA,
            pltpu.SemaphoreType.DMA,
        ],
    )
    def gk(table_hbm, idx_hbm, out_hbm, idx_v, rows_v, sem_g, sem_w):
        wid = lax.axis_index("s") * 2 + lax.axis_index("c")
        e0 = wid * epw
        pltpu.sync_copy(idx_hbm.at[pl.ds(e0, epw)], idx_v)
        pltpu.async_copy(
            table_hbm.at[idx_v.at[pl.ds(0, chunk)]], rows_v.at[0], sem_g)

        def body(j, carry):
            b = lax.rem(j, 2)
            pltpu.make_async_copy(
                table_hbm.at[idx_v.at[pl.ds(0, chunk)]], rows_v.at[b], sem_g
            ).wait()

            @pl.when(j < steps - 1)
            def _fire_next():
                @pl.when(j >= 1)
                def _drain_prev_write():
                    pltpu.make_async_copy(
                        rows_v.at[1 - b], out_hbm.at[pl.ds(0, chunk)], sem_w
                    ).wait()
                pltpu.async_copy(
                    table_hbm.at[idx_v.at[pl.ds((j + 1) * chunk, chunk)]],
                    rows_v.at[1 - b], sem_g)

            pltpu.async_copy(
                rows_v.at[b], out_hbm.at[pl.ds(e0 + j * chunk, chunk)], sem_w)
            return carry

        lax.fori_loop(0, steps, body, 0)
        for _ in range(2):
            pltpu.make_async_copy(
                rows_v.at[0], out_hbm.at[pl.ds(0, chunk)], sem_w).wait()

    return gk(table, idx)


def _entry_body(x_ref, gp_ref, gpe_ref, w1_ref, b1_ref, w2_ref, bnw_ref,
                bnb_ref, proj_ref, xo_ref, xp_ref):
    x = x_ref[...] + _dot(gp_ref[...], gpe_ref[...])
    h = _gelu(_dot(x, w1_ref[...]) + b1_ref[...])
    x = x + _dot(h, w2_ref[...]) * bnw_ref[...] + bnb_ref[...]
    xo_ref[...] = x
    xp_ref[...] = _dot(x.astype(jnp.bfloat16), proj_ref[...])


def _lfa_body(nk, first, has_mlp, has_next, xyzin_ref, xpg_ref, x_ref, xp_ref,
              *rest):
    i = 0
    if first:
        xyzc_ref = rest[0]
        i = 1
    (w1_ref, w2_ref, b2_ref, w3at_ref, w3ab_ref, b3a_ref, w3b_ref,
     b3b_ref, bnw_ref, bnb_ref) = rest[i:i + 10]
    i += 10
    if has_mlp:
        mw1_ref, mb1_ref, mw2_ref, mbnw_ref, mbnb_ref = rest[i:i + 5]
        i += 5
    if has_next:
        projn_ref = rest[i]
        i += 1
    xo_ref = rest[i]
    i += 1
    if has_next:
        xpn_ref = rest[i]
        i += 1
    if first:
        xyzs_out_ref = rest[i]

    B = x_ref.shape[0]
    D2 = w2_ref.shape[0]
    D = w3b_ref.shape[0]

    bf16 = jnp.bfloat16
    f32 = jnp.float32
    if first:
        xyzs3 = xyzin_ref[...][:, :, :16] - xyzc_ref[...][None, :, :]
        xyzs3 = xyzs3.astype(bf16)
        xyzs_out_ref[...] = xyzs3
    else:
        xyzs3 = xyzin_ref[...]
    xyzs = xyzs3.reshape(nk * B, 16)
    p0 = _dot(xyzs, w1_ref[...])
    plocal = jnp.max(p0.reshape(nk, B, D2), axis=0)
    p1 = _dot(p0.astype(bf16), w2_ref[...]) + b2_ref[...]
    u = _dot(plocal.astype(bf16), w3ab_ref[...]) + b3a_ref[...]
    v = _dot(p1.astype(bf16), w3at_ref[...])
    h = _gelu(v.reshape(nk, B, D) + u[None, :, :]).reshape(nk * B, D)
    s = xpg_ref[...] + _dot(h.astype(bf16), w3b_ref[...]).reshape(nk, B, D)
    xm = jnp.max(s, axis=0) - xp_ref[...] + b3b_ref[...]
    xn = x_ref[...] + xm * bnw_ref[...] + bnb_ref[...]
    if has_mlp:
        t = _gelu(_dot(xn.astype(bf16), mw1_ref[...]) + mb1_ref[...])
        xn = (xn + _dot(t.astype(bf16), mw2_ref[...]) * mbnw_ref[...]
              + mbnb_ref[...])
    xo_ref[...] = xn
    if has_next:
        xpn_ref[...] = _dot(xn.astype(bf16), projn_ref[...])


def _row_spec(r, c):
    return pl.BlockSpec((r, c), lambda b: (b, 0))


def _full_spec(r, c):
    return pl.BlockSpec((r, c), lambda b: (0, 0))


def kernel(xyz, x, knn, g_pos, gpe_W, mlp_W1, mlp_b1, mlp_W2, mlp_bn_w,
           mlp_bn_b, lfa_proj_W, lfa_bn_w, lfa_bn_b, nca1_W, nca1_b, nca2_W,
           nca2_b, nca3a_W, nca3a_b, nca3b_W, nca3b_b, mlps_W1, mlps_b1,
           mlps_W2, mlps_bn_w, mlps_bn_b):
    f32 = jnp.float32
    n, d = x.shape
    nk = knn.shape[1]
    hid = mlp_W1.shape[0]
    d2 = nca1_W.shape[1]
    depth = lfa_proj_W.shape[0]
    gdim = g_pos.shape[1]

    ne = n * nk
    knn_flat = knn.astype(jnp.int32).T.reshape(-1)

    bf16 = jnp.bfloat16
    xyz128 = jnp.concatenate(
        [xyz.astype(f32), jnp.zeros((n, 12), f32), jnp.ones((n, 1), f32),
         jnp.zeros((n, d - 16), f32)], axis=1)
    xyzc16 = jnp.concatenate([xyz.astype(f32), jnp.zeros((n, 13), f32)],
                             axis=1)

    b0 = 1000
    entry = pl.pallas_call(
        _entry_body,
        grid=(n // b0,),
        in_specs=[
            _row_spec(b0, d), _row_spec(b0, gdim), _full_spec(gdim, d),
            _full_spec(d, hid), _full_spec(1, hid), _full_spec(hid, d),
            _full_spec(1, d), _full_spec(1, d), _full_spec(d, d),
        ],
        out_specs=[_row_spec(b0, d), _row_spec(b0, d)],
        out_shape=[jax.ShapeDtypeStruct((n, d), f32)] * 2,
    )
    xcur, xp = entry(x.astype(f32), g_pos, gpe_W.T, mlp_W1.T, mlp_b1[None],
                     mlp_W2.T, (RS * mlp_bn_w)[None], mlp_bn_b[None],
                     lfa_proj_W[0].T.astype(bf16))

    split = 2
    nh = n // split
    bb = 200
    nblk = nh // bb
    epw = nh * nk // _NW
    _cands = [c for c in range(8, 401, 8) if epw % c == 0]
    gchunk = max(_cands) if _cands else 8
    idx_h = [knn[h * nh:(h + 1) * nh].astype(jnp.int32).T.reshape(-1)
             for h in range(split)]

    xyzg = [_sc_gather(xyz128, idx_h[h], gchunk).reshape(nk, nh, d)
            for h in range(split)]
    xyzs16 = [None] * split

    for i in range(depth):
        xpg = [_sc_gather(xp, idx_h[h], gchunk).reshape(nk, nh, d)
               for h in range(split)]
        first = (i == 0)
        has_mlp = (i % 2 == 1)
        has_next = (i < depth - 1)
        w1p = jnp.concatenate([nca1_W[i].T, jnp.zeros((12, d2), f32),
                               nca1_b[i][None]], axis=0).astype(bf16)

        new_x, new_xp = [], []
        for h in range(split):
            off = h * nblk
            hspec = lambda r, c, o=off: pl.BlockSpec(
                (r, c), lambda b, o=o: (b + o, 0))
            args = [xyzg[h] if first else xyzs16[h], xpg[h], xcur, xp]
            in_specs = [
                pl.BlockSpec((nk, bb, d if first else 16),
                             lambda b: (0, b, 0)),
                pl.BlockSpec((nk, bb, d), lambda b: (0, b, 0)),
                hspec(bb, d), hspec(bb, d),
            ]
            if first:
                args.append(xyzc16)
                in_specs.append(hspec(bb, 16))
            args += [w1p, nca2_W[i].T.astype(bf16),
                     nca2_b[i][None], nca3a_W[i][:, :d2].T.astype(bf16),
                     nca3a_W[i][:, d2:].T.astype(bf16), nca3a_b[i][None],
                     nca3b_W[i].T.astype(bf16), nca3b_b[i][None],
                     (RS * lfa_bn_w[i])[None], lfa_bn_b[i][None]]
            in_specs += [
                _full_spec(16, d2), _full_spec(d2, d2),
                _full_spec(1, d2), _full_spec(d2, d), _full_spec(d2, d),
                _full_spec(1, d), _full_spec(d, d), _full_spec(1, d),
                _full_spec(1, d), _full_spec(1, d),
            ]
            if has_mlp:
                j = i // 2
                args += [mlps_W1[j].T.astype(bf16), mlps_b1[j][None],
                         mlps_W2[j].T.astype(bf16),
                         (RS * mlps_bn_w[j])[None], mlps_bn_b[j][None]]
                in_specs += [_full_spec(d, hid), _full_spec(1, hid),
                             _full_spec(hid, d), _full_spec(1, d),
                             _full_spec(1, d)]
            if has_next:
                args.append(lfa_proj_W[i + 1].T.astype(bf16))
                in_specs.append(_full_spec(d, d))

            out_specs = [_row_spec(bb, d)]
            out_shape = [jax.ShapeDtypeStruct((nh, d), f32)]
            if has_next:
                out_specs.append(_row_spec(bb, d))
                out_shape.append(jax.ShapeDtypeStruct((nh, d), f32))
            if first:
                out_specs.append(
                    pl.BlockSpec((nk, bb, 16), lambda b: (0, b, 0)))
                out_shape.append(jax.ShapeDtypeStruct((nk, nh, 16), bf16))

            outs = pl.pallas_call(
                functools.partial(_lfa_body, nk, first, has_mlp, has_next),
                grid=(nblk,),
                in_specs=in_specs,
                out_specs=out_specs,
                out_shape=out_shape,
            )(*args)
            new_x.append(outs[0])
            if has_next:
                new_xp.append(outs[1])
            if first:
                xyzs16[h] = outs[2]

        xcur = jnp.concatenate(new_x, axis=0)
        if has_next:
            xp = jnp.concatenate(new_xp, axis=0)

    return xcur


# speedup vs baseline: 1.0160x; 1.0160x over previous
import functools
import math

import jax
import jax.numpy as jnp
from jax import lax
from jax.experimental import pallas as pl
from jax.experimental.pallas import tpu as pltpu
from jax.experimental.pallas import tpu_sc as plsc

EPS = 1e-5
RS = 1.0 / math.sqrt(1.0 + EPS)
_SQRT_HALF = 1.0 / math.sqrt(2.0)

_NW = 32
_GROW = 128


def _gelu(v):
    return 0.5 * v * (1.0 + lax.erf(v * _SQRT_HALF))


def _dot(a, b):
    return jnp.dot(a, b, preferred_element_type=jnp.float32)


def _sc_gather(table, idx, chunk=400):
    total = idx.shape[0]
    width = table.shape[1]
    dtype = table.dtype
    epw = total // _NW
    assert epw % chunk == 0 and chunk % 8 == 0 and epw % 8 == 0
    steps = epw // chunk

    mesh = plsc.VectorSubcoreMesh(core_axis_name="c", subcore_axis_name="s")

    @functools.partial(
        pl.kernel,
        out_type=jax.ShapeDtypeStruct((total, width), dtype),
        mesh=mesh,
        scratch_types=[
            pltpu.VMEM((epw,), jnp.int32),
            pltpu.VMEM((3, chunk, width), dtype),
            pltpu.SemaphoreType.DMA,
            pltpu.SemaphoreType.DMA,
        ],
    )
    def gk(table_hbm, idx_hbm, out_hbm, idx_v, rows_v, sem_g, sem_w):
        wid = lax.axis_index("s") * 2 + lax.axis_index("c")
        e0 = wid * epw
        pltpu.sync_copy(idx_hbm.at[pl.ds(e0, epw)], idx_v)
        pltpu.async_copy(
            table_hbm.at[idx_v.at[pl.ds(0, chunk)]], rows_v.at[0], sem_g)
        pltpu.async_copy(
            table_hbm.at[idx_v.at[pl.ds(chunk, chunk)]], rows_v.at[1], sem_g)

        def body(j, carry):
            b = lax.rem(j, 3)
            pltpu.make_async_copy(
                table_hbm.at[idx_v.at[pl.ds(0, chunk)]], rows_v.at[b], sem_g
            ).wait()

            @pl.when(j < steps - 2)
            def _fire_next():
                bn = lax.rem(j + 2, 3)

                @pl.when(j >= 1)
                def _drain_prev_write():
                    pltpu.make_async_copy(
                        rows_v.at[bn], out_hbm.at[pl.ds(0, chunk)], sem_w
                    ).wait()
                pltpu.async_copy(
                    table_hbm.at[idx_v.at[pl.ds((j + 2) * chunk, chunk)]],
                    rows_v.at[bn], sem_g)

            pltpu.async_copy(
                rows_v.at[b], out_hbm.at[pl.ds(e0 + j * chunk, chunk)], sem_w)
            return carry

        lax.fori_loop(0, steps, body, 0)
        for _ in range(3):
            pltpu.make_async_copy(
                rows_v.at[0], out_hbm.at[pl.ds(0, chunk)], sem_w).wait()

    return gk(table, idx)


def _entry_body(x_ref, gp_ref, gpe_ref, w1_ref, b1_ref, w2_ref, bnw_ref,
                bnb_ref, proj_ref, xo_ref, xp_ref):
    x = x_ref[...] + _dot(gp_ref[...], gpe_ref[...])
    h = _gelu(_dot(x, w1_ref[...]) + b1_ref[...])
    x = x + _dot(h, w2_ref[...]) * bnw_ref[...] + bnb_ref[...]
    xo_ref[...] = x
    xp_ref[...] = _dot(x.astype(jnp.bfloat16), proj_ref[...])


def _lfa_body(nk, first, has_mlp, has_next, xyzin_ref, xpg_ref, x_ref, xp_ref,
              *rest):
    i = 0
    if first:
        xyzc_ref = rest[0]
        i = 1
    (w1_ref, w2_ref, b2_ref, w3at_ref, w3ab_ref, b3a_ref, w3b_ref,
     b3b_ref, bnw_ref, bnb_ref) = rest[i:i + 10]
    i += 10
    if has_mlp:
        mw1_ref, mb1_ref, mw2_ref, mbnw_ref, mbnb_ref = rest[i:i + 5]
        i += 5
    if has_next:
        projn_ref = rest[i]
        i += 1
    xo_ref = rest[i]
    i += 1
    if has_next:
        xpn_ref = rest[i]
        i += 1
    if first:
        xyzs_out_ref = rest[i]

    B = x_ref.shape[0]
    D2 = w2_ref.shape[0]
    D = w3b_ref.shape[0]

    bf16 = jnp.bfloat16
    f32 = jnp.float32
    if first:
        xyzs3 = xyzin_ref[...][:, :, :16] - xyzc_ref[...][None, :, :]
        xyzs3 = xyzs3.astype(bf16)
        xyzs_out_ref[...] = xyzs3
    else:
        xyzs3 = xyzin_ref[...]
    xyzs = xyzs3.reshape(nk * B, 16)
    p0 = _dot(xyzs, w1_ref[...])
    plocal = jnp.max(p0.reshape(nk, B, D2), axis=0)
    p1 = _dot(p0.astype(bf16), w2_ref[...]) + b2_ref[...]
    u = _dot(plocal.astype(bf16), w3ab_ref[...]) + b3a_ref[...]
    v = _dot(p1.astype(bf16), w3at_ref[...])
    h = _gelu(v.reshape(nk, B, D) + u[None, :, :]).reshape(nk * B, D)
    s = xpg_ref[...] + _dot(h.astype(bf16), w3b_ref[...]).reshape(nk, B, D)
    xm = jnp.max(s, axis=0) - xp_ref[...] + b3b_ref[...]
    xn = x_ref[...] + xm * bnw_ref[...] + bnb_ref[...]
    if has_mlp:
        t = _gelu(_dot(xn.astype(bf16), mw1_ref[...]) + mb1_ref[...])
        xn = (xn + _dot(t.astype(bf16), mw2_ref[...]) * mbnw_ref[...]
              + mbnb_ref[...])
    xo_ref[...] = xn
    if has_next:
        xpn_ref[...] = _dot(xn.astype(bf16), projn_ref[...])


def _row_spec(r, c):
    return pl.BlockSpec((r, c), lambda b: (b, 0))


def _full_spec(r, c):
    return pl.BlockSpec((r, c), lambda b: (0, 0))


def kernel(xyz, x, knn, g_pos, gpe_W, mlp_W1, mlp_b1, mlp_W2, mlp_bn_w,
           mlp_bn_b, lfa_proj_W, lfa_bn_w, lfa_bn_b, nca1_W, nca1_b, nca2_W,
           nca2_b, nca3a_W, nca3a_b, nca3b_W, nca3b_b, mlps_W1, mlps_b1,
           mlps_W2, mlps_bn_w, mlps_bn_b):
    f32 = jnp.float32
    n, d = x.shape
    nk = knn.shape[1]
    hid = mlp_W1.shape[0]
    d2 = nca1_W.shape[1]
    depth = lfa_proj_W.shape[0]
    gdim = g_pos.shape[1]

    ne = n * nk
    knn_flat = knn.astype(jnp.int32).T.reshape(-1)

    bf16 = jnp.bfloat16
    xyz128 = jnp.concatenate(
        [xyz.astype(f32), jnp.zeros((n, 12), f32), jnp.ones((n, 1), f32),
         jnp.zeros((n, d - 16), f32)], axis=1)
    xyzc16 = jnp.concatenate([xyz.astype(f32), jnp.zeros((n, 13), f32)],
                             axis=1)

    b0 = 1000
    entry = pl.pallas_call(
        _entry_body,
        grid=(n // b0,),
        in_specs=[
            _row_spec(b0, d), _row_spec(b0, gdim), _full_spec(gdim, d),
            _full_spec(d, hid), _full_spec(1, hid), _full_spec(hid, d),
            _full_spec(1, d), _full_spec(1, d), _full_spec(d, d),
        ],
        out_specs=[_row_spec(b0, d), _row_spec(b0, d)],
        out_shape=[jax.ShapeDtypeStruct((n, d), f32)] * 2,
    )
    xcur, xp = entry(x.astype(f32), g_pos, gpe_W.T, mlp_W1.T, mlp_b1[None],
                     mlp_W2.T, (RS * mlp_bn_w)[None], mlp_bn_b[None],
                     lfa_proj_W[0].T.astype(bf16))

    split = 2
    nh = n // split
    bb = 200
    nblk = nh // bb
    epw = nh * nk // _NW
    _cands = [c for c in range(8, 401, 8) if epw % c == 0]
    gchunk = max(_cands) if _cands else 8
    idx_h = [knn[h * nh:(h + 1) * nh].astype(jnp.int32).T.reshape(-1)
             for h in range(split)]

    xyzg = [_sc_gather(xyz128, idx_h[h], gchunk).reshape(nk, nh, d)
            for h in range(split)]
    xyzs16 = [None] * split

    for i in range(depth):
        xpg = [_sc_gather(xp, idx_h[h], gchunk).reshape(nk, nh, d)
               for h in range(split)]
        first = (i == 0)
        has_mlp = (i % 2 == 1)
        has_next = (i < depth - 1)
        w1p = jnp.concatenate([nca1_W[i].T, jnp.zeros((12, d2), f32),
                               nca1_b[i][None]], axis=0).astype(bf16)

        new_x, new_xp = [], []
        for h in range(split):
            off = h * nblk
            hspec = lambda r, c, o=off: pl.BlockSpec(
                (r, c), lambda b, o=o: (b + o, 0))
            args = [xyzg[h] if first else xyzs16[h], xpg[h], xcur, xp]
            in_specs = [
                pl.BlockSpec((nk, bb, d if first else 16),
                             lambda b: (0, b, 0)),
                pl.BlockSpec((nk, bb, d), lambda b: (0, b, 0)),
                hspec(bb, d), hspec(bb, d),
            ]
            if first:
                args.append(xyzc16)
                in_specs.append(hspec(bb, 16))
            args += [w1p, nca2_W[i].T.astype(bf16),
                     nca2_b[i][None], nca3a_W[i][:, :d2].T.astype(bf16),
                     nca3a_W[i][:, d2:].T.astype(bf16), nca3a_b[i][None],
                     nca3b_W[i].T.astype(bf16), nca3b_b[i][None],
                     (RS * lfa_bn_w[i])[None], lfa_bn_b[i][None]]
            in_specs += [
                _full_spec(16, d2), _full_spec(d2, d2),
                _full_spec(1, d2), _full_spec(d2, d), _full_spec(d2, d),
                _full_spec(1, d), _full_spec(d, d), _full_spec(1, d),
                _full_spec(1, d), _full_spec(1, d),
            ]
            if has_mlp:
                j = i // 2
                args += [mlps_W1[j].T.astype(bf16), mlps_b1[j][None],
                         mlps_W2[j].T.astype(bf16),
                         (RS * mlps_bn_w[j])[None], mlps_bn_b[j][None]]
                in_specs += [_full_spec(d, hid), _full_spec(1, hid),
                             _full_spec(hid, d), _full_spec(1, d),
                             _full_spec(1, d)]
            if has_next:
                args.append(lfa_proj_W[i + 1].T.astype(bf16))
                in_specs.append(_full_spec(d, d))

            out_specs = [_row_spec(bb, d)]
            out_shape = [jax.ShapeDtypeStruct((nh, d), f32)]
            if has_next:
                out_specs.append(_row_spec(bb, d))
                out_shape.append(jax.ShapeDtypeStruct((nh, d), f32))
            if first:
                out_specs.append(
                    pl.BlockSpec((nk, bb, 16), lambda b: (0, b, 0)))
                out_shape.append(jax.ShapeDtypeStruct((nk, nh, 16), bf16))

            outs = pl.pallas_call(
                functools.partial(_lfa_body, nk, first, has_mlp, has_next),
                grid=(nblk,),
                in_specs=in_specs,
                out_specs=out_specs,
                out_shape=out_shape,
            )(*args)
            new_x.append(outs[0])
            if has_next:
                new_xp.append(outs[1])
            if first:
                xyzs16[h] = outs[2]

        xcur = jnp.concatenate(new_x, axis=0)
        if has_next:
            xp = jnp.concatenate(new_xp, axis=0)

    return xcur
